# baseline (speedup 1.0000x reference)
import jax
import jax.numpy as jnp
from jax import lax
from jax.experimental import pallas as pl
from jax.experimental.pallas import tpu as pltpu
from jax.experimental.pallas import tpu_sc as plsc

NC = 2
NS = 16
NW = NC * NS
C = 128
NBUF = 2
L = 16
SHIFT = 14


def _sc_agg_kernel(n_pad, k, d, interpret=False):
    rps = n_pad // NS
    n_outer = k // NBUF

    def body(x_hbm, pk_hbm, z_hbm, out_hbm,
             agg_sh, pk_v, src_v, dst_v, gbuf, gsem):
        cid = lax.axis_index("c")
        sid = lax.axis_index("s")
        wid = sid * NC + cid

        pltpu.sync_copy(z_hbm, agg_sh.at[pl.ds(sid * rps, rps)])
        pltpu.sync_copy(pk_hbm.at[wid], pk_v)
        plsc.subcore_barrier()

        def unpack(j, b):
            for i in range(C // L):
                pk = pk_v[j, pl.ds(i * L, L)]
                src_v[b, pl.ds(i * L, L)] = lax.shift_right_logical(pk, SHIFT)
                dst_v[b, pl.ds(i * L, L)] = lax.bitwise_and(pk, (1 << SHIFT) - 1)

        for b in range(NBUF):
            unpack(b, b)
            pltpu.async_copy(x_hbm.at[src_v.at[b]], gbuf.at[b], gsem.at[b])

        def outer(g, carry):
            for b in range(NBUF):
                j = g * NBUF + b
                pltpu.make_async_copy(
                    x_hbm.at[src_v.at[b]], gbuf.at[b], gsem.at[b]).wait()
                pltpu.sync_copy(gbuf.at[b], agg_sh.at[dst_v.at[b]], add=True)

                @pl.when(g < n_outer - 1)
                def _():
                    unpack(j + NBUF, b)
                    pltpu.async_copy(
                        x_hbm.at[src_v.at[b]], gbuf.at[b], gsem.at[b])
            return carry

        lax.fori_loop(0, n_outer, outer, 0)
        plsc.subcore_barrier()
        pltpu.sync_copy(agg_sh.at[pl.ds(sid * rps, rps)],
                        out_hbm.at[cid, pl.ds(sid * rps, rps)])

    mesh = plsc.VectorSubcoreMesh(core_axis_name="c", subcore_axis_name="s")
    return pl.kernel(
        body,
        out_type=jax.ShapeDtypeStruct((NC, n_pad, d), jnp.float32),
        mesh=mesh,
        scratch_types=[
            pltpu.VMEM_SHARED((n_pad, d), jnp.float32),
            pltpu.VMEM((k, C), jnp.int32),
            pltpu.VMEM((NBUF, C), jnp.int32),
            pltpu.VMEM((NBUF, C), jnp.int32),
            pltpu.VMEM((NBUF, C, d), jnp.float32),
            pltpu.SemaphoreType.DMA((NBUF,)),
        ],
        interpret=interpret,
    )


def _tc_body(p0_ref, p1_ref, x_ref, wt_ref, o_ref):
    agg = p0_ref[...] + p1_ref[...]
    h = jnp.dot(agg, wt_ref[...], preferred_element_type=jnp.float32)
    o_ref[...] = jnp.maximum(h, 0.0) + x_ref[...]


@jax.jit
def kernel(x, edge_index, W):
    n, d = x.shape
    e = edge_index.shape[1]

    k = -(-e // (NW * C * NBUF)) * NBUF
    e_pad = NW * k * C
    n_pad = -(-(n + 1) // (NS * 8)) * (NS * 8)

    src = edge_index[0]
    dst = edge_index[1]
    pad_dst = n + (jnp.arange(e_pad - e, dtype=jnp.int32) % (n_pad - n))
    src_p = jnp.concatenate([src, jnp.zeros((e_pad - e,), jnp.int32)])
    dst_p = jnp.concatenate([dst, pad_dst])
    packed = ((src_p << SHIFT) | dst_p).reshape(NW, k, C)
    zrows = jnp.zeros((n_pad // NS, d), jnp.float32)

    partials = _sc_agg_kernel(n_pad, k, d)(x, packed, zrows)

    nb = 8 * 125
    out = pl.pallas_call(
        _tc_body,
        out_shape=jax.ShapeDtypeStruct((n, d), jnp.float32),
        grid=(n // nb,),
        in_specs=[
            pl.BlockSpec((nb, d), lambda i: (i, 0)),
            pl.BlockSpec((nb, d), lambda i: (i, 0)),
            pl.BlockSpec((nb, d), lambda i: (i, 0)),
            pl.BlockSpec((d, d), lambda i: (0, 0)),
        ],
        out_specs=pl.BlockSpec((nb, d), lambda i: (i, 0)),
    )(partials[0, :n], partials[1, :n], x, W.T)
    return out
